# Initial kernel scaffold; baseline (speedup 1.0000x reference)
import jax
import jax.numpy as jnp
from jax.experimental import pallas as pl


def kernel(x, action_type, W1, b1, W2, b2, g):
    raise NotImplementedError("write your pallas kernel here")


# speedup vs baseline: 2.5466x; 2.5466x over previous
import functools

import jax
import jax.numpy as jnp
from jax.experimental import pallas as pl
from jax.experimental.pallas import tpu as pltpu

N = 2048
D = 768
H = 2048
E = 8
EPS = 1e-06

B = 256
NB = N // B


def _mlp_body(off_ref, x_ref, W1_ref, b1_ref, W2_ref, b2_ref, g_ref, out_ref):
    e = pl.program_id(0)
    nb = pl.program_id(1)
    start = off_ref[e]
    end = off_ref[e + 1]
    blk_lo = nb * B

    @pl.when((start < blk_lo + B) & (end > blk_lo))
    def _():
        x = x_ref[...]
        p = jax.lax.dot_general(
            x, W1_ref[0], (((1,), (1,)), ((), ())),
            preferred_element_type=jnp.float32)
        p = p + b1_ref[0]
        proj = p[:, :H]
        gate = p[:, H:]
        h = proj * (gate * jax.lax.logistic(gate))
        y = jax.lax.dot_general(
            h, W2_ref[0], (((1,), (1,)), ((), ())),
            preferred_element_type=jnp.float32)
        y = y + b2_ref[0]
        z = x + y
        ms = jnp.mean(z * z, axis=-1, keepdims=True)
        z = z * jax.lax.rsqrt(ms + EPS) * g_ref[0]
        gid = blk_lo + jax.lax.broadcasted_iota(jnp.int32, (B, 1), 0)
        mask = (gid >= start) & (gid < end)
        cur = out_ref[pl.ds(blk_lo, B), :]
        out_ref[pl.ds(blk_lo, B), :] = jnp.where(mask, z, cur)


def _x_map(e, nb, off):
    start = off[e]
    end = off[e + 1]
    lo = start // B
    hi = jnp.where(end > start, (end - 1) // B, lo)
    return (jnp.clip(nb, lo, hi), 0)


@functools.partial(jax.jit, static_argnames=("interpret",))
def _mlp_sorted(offsets, x_sorted, W1, b1, W2, b2, g, interpret=False):
    grid_spec = pltpu.PrefetchScalarGridSpec(
        num_scalar_prefetch=1,
        grid=(E, NB),
        in_specs=[
            pl.BlockSpec((B, D), _x_map),
            pl.BlockSpec((1, 2 * H, D), lambda e, nb, off: (e, 0, 0)),
            pl.BlockSpec((1, 1, 2 * H), lambda e, nb, off: (e, 0, 0)),
            pl.BlockSpec((1, D, H), lambda e, nb, off: (e, 0, 0)),
            pl.BlockSpec((1, 1, D), lambda e, nb, off: (e, 0, 0)),
            pl.BlockSpec((1, D), lambda e, nb, off: (0, 0)),
        ],
        out_specs=pl.BlockSpec((N, D), lambda e, nb, off: (0, 0)),
    )
    return pl.pallas_call(
        _mlp_body,
        grid_spec=grid_spec,
        out_shape=jax.ShapeDtypeStruct((N, D), jnp.float32),
        compiler_params=pltpu.CompilerParams(
            dimension_semantics=("arbitrary", "arbitrary")),
        interpret=interpret,
    )(offsets, x_sorted,
      W1, b1.reshape(E, 1, 2 * H), W2, b2.reshape(E, 1, D),
      g.reshape(1, D))


def kernel(x, action_type, W1, b1, W2, b2, g, interpret=False):
    at = action_type.astype(jnp.int32)
    sort_idx = jnp.argsort(at)
    counts = jnp.bincount(at, length=E)
    offsets = jnp.concatenate(
        [jnp.zeros((1,), jnp.int32), jnp.cumsum(counts).astype(jnp.int32)])
    x_sorted = jnp.take(x, sort_idx, axis=0)
    z_sorted = _mlp_sorted(offsets, x_sorted, W1, b1, W2, b2, g,
                           interpret=interpret)
    return jnp.zeros_like(x).at[sort_idx].set(z_sorted)
